# baseline (speedup 1.0000x reference)
import jax
import jax.numpy as jnp
from jax.experimental import pallas as pl
from jax.experimental.pallas import tpu as pltpu

_H = 50
_W = 50
_C = 128
_K = 100
_PH = 7
_SCALE = 0.25
_SAMP = 16
_WIN = 24


def _interp_weights(c0, bsz, n, base):
    g = base + jax.lax.broadcasted_iota(jnp.int32, (n, _SAMP), 0).astype(jnp.float32)
    si = jax.lax.broadcasted_iota(jnp.int32, (n, _SAMP), 1).astype(jnp.float32)
    pbin = jnp.floor(si * 0.5)
    off = (si - 2.0 * pbin + 0.5) * 0.5
    coord = c0 + (pbin + off) * bsz
    valid = (coord >= -1.0) & (coord <= 50.0) & (si < float(_PH * 2))
    cc = jnp.maximum(coord, 0.0)
    cl = jnp.minimum(jnp.floor(cc), 49.0)
    ch = jnp.minimum(cl + 1.0, 49.0)
    cv = jnp.where(cl >= 49.0, cl, cc)
    lf = cv - cl
    hf = 1.0 - lf
    w = (g == cl).astype(jnp.float32) * hf + (g == ch).astype(jnp.float32) * lf
    return w * valid.astype(jnp.float32)


def _ybase(boxes_ref, i):
    y1 = boxes_ref[i, 2] * _SCALE
    y0 = jnp.floor(jnp.clip(y1, 0.0, 49.0)).astype(jnp.int32)
    return jnp.minimum(y0, _H - _WIN)


def _roi_kernel(boxes_ref, feat_ref, mask_hbm, out_ref, mwin_ref, sem_ref):
    k = pl.program_id(0)

    _NSPLIT = 4
    _CSPL = _C // _NSPLIT

    def window_copies(i, slot):
        ya = _ybase(boxes_ref, i)
        return [
            pltpu.make_async_copy(
                mask_hbm.at[i, pl.ds(j * _CSPL, _CSPL), pl.ds(ya, _WIN), :],
                mwin_ref.at[slot, pl.ds(j * _CSPL, _CSPL)],
                sem_ref.at[slot, j],
            )
            for j in range(_NSPLIT)
        ]

    @pl.when(k == 0)
    def _():
        for c in window_copies(0, 0):
            c.start()

    @pl.when(k + 1 < _K)
    def _():
        for c in window_copies(k + 1, (k + 1) % 2):
            c.start()

    for c in window_copies(k, k % 2):
        c.wait()

    b = boxes_ref[k, 0].astype(jnp.int32)
    x1 = boxes_ref[k, 1] * _SCALE
    y1 = boxes_ref[k, 2] * _SCALE
    x2 = boxes_ref[k, 3] * _SCALE
    y2 = boxes_ref[k, 4] * _SCALE
    bw = jnp.maximum(x2 - x1, 1.0) / float(_PH)
    bh = jnp.maximum(y2 - y1, 1.0) / float(_PH)
    ya = _ybase(boxes_ref, k)

    wx = _interp_weights(x1, bw, _W, 0.0)
    wy = _interp_weights(y1, bh, _WIN, ya.astype(jnp.float32))

    f = feat_ref[b, :, pl.ds(ya, _WIN), :]
    mf = f * mwin_ref[k % 2]

    a = jax.lax.dot_general(
        mf.reshape(_C * _WIN, _W), wx, (((1,), (0,)), ((), ())),
        preferred_element_type=jnp.float32).reshape(_C, _WIN, _SAMP)

    u = jax.lax.dot_general(
        a, wy, (((1,), (0,)), ((), ())),
        preferred_element_type=jnp.float32)

    s_ = jax.lax.broadcasted_iota(jnp.int32, (_SAMP * _SAMP, 56), 0)
    r_ = jax.lax.broadcasted_iota(jnp.int32, (_SAMP * _SAMP, 56), 1)
    bigpool = (((s_ % _SAMP) // 2 == r_ // _PH)
               & ((s_ // _SAMP) // 2 == r_ % _PH)).astype(jnp.float32) * 0.25

    w = jax.lax.dot_general(
        u.reshape(_C, _SAMP * _SAMP), bigpool, (((1,), (0,)), ((), ())),
        preferred_element_type=jnp.float32)
    out_ref[0] = w[:, :_PH * _PH]


@jax.jit
def kernel(features, boxes, masks):
    out = pl.pallas_call(
        _roi_kernel,
        grid=(_K,),
        in_specs=[
            pl.BlockSpec(memory_space=pltpu.SMEM),
            pl.BlockSpec((2, _C, _H, _W), lambda k: (0, 0, 0, 0)),
            pl.BlockSpec(memory_space=pltpu.MemorySpace.HBM),
        ],
        out_specs=pl.BlockSpec((1, _C, _PH * _PH), lambda k: (k, 0, 0)),
        out_shape=jax.ShapeDtypeStruct((_K, _C, _PH * _PH), jnp.float32),
        scratch_shapes=[
            pltpu.VMEM((2, _C, _WIN, _W), jnp.float32),
            pltpu.SemaphoreType.DMA((2, 4)),
        ],
        compiler_params=pltpu.CompilerParams(
            dimension_semantics=("arbitrary",),
        ),
    )(boxes, features, masks)
    return out.reshape(_K, _C, _PH, _PH)


# speedup vs baseline: 1.0764x; 1.0764x over previous
import jax
import jax.numpy as jnp
from jax.experimental import pallas as pl
from jax.experimental.pallas import tpu as pltpu

_H = 50
_W = 50
_C = 128
_K = 100
_PH = 7
_SCALE = 0.25
_SAMP = 16
_WIN = 24
_NBUF = 4


def _interp_weights(c0, bsz, n, base):
    g = base + jax.lax.broadcasted_iota(jnp.int32, (n, _SAMP), 0).astype(jnp.float32)
    si = jax.lax.broadcasted_iota(jnp.int32, (n, _SAMP), 1).astype(jnp.float32)
    pbin = jnp.floor(si * 0.5)
    off = (si - 2.0 * pbin + 0.5) * 0.5
    coord = c0 + (pbin + off) * bsz
    valid = (coord >= -1.0) & (coord <= 50.0) & (si < float(_PH * 2))
    cc = jnp.maximum(coord, 0.0)
    cl = jnp.minimum(jnp.floor(cc), 49.0)
    ch = jnp.minimum(cl + 1.0, 49.0)
    cv = jnp.where(cl >= 49.0, cl, cc)
    lf = cv - cl
    hf = 1.0 - lf
    w = (g == cl).astype(jnp.float32) * hf + (g == ch).astype(jnp.float32) * lf
    return w * valid.astype(jnp.float32)


def _ybase(boxes_ref, i):
    y1 = boxes_ref[i, 2] * _SCALE
    y0 = jnp.floor(jnp.clip(y1, 0.0, 49.0)).astype(jnp.int32)
    return jnp.minimum(y0, _H - _WIN)


def _roi_kernel(boxes_ref, feat_ref, mask_hbm, out_ref, mwin_ref, sem_ref):
    k = pl.program_id(0)

    def window_copy(i, slot):
        ya = _ybase(boxes_ref, i)
        return pltpu.make_async_copy(
            mask_hbm.at[i, :, pl.ds(ya, _WIN), :],
            mwin_ref.at[slot],
            sem_ref.at[slot],
        )

    @pl.when(k == 0)
    def _():
        for i in range(_NBUF - 1):
            window_copy(i, i).start()

    @pl.when(k + _NBUF - 1 < _K)
    def _():
        window_copy(k + _NBUF - 1, (k + _NBUF - 1) % _NBUF).start()

    window_copy(k, k % _NBUF).wait()

    b = boxes_ref[k, 0].astype(jnp.int32)
    x1 = boxes_ref[k, 1] * _SCALE
    y1 = boxes_ref[k, 2] * _SCALE
    x2 = boxes_ref[k, 3] * _SCALE
    y2 = boxes_ref[k, 4] * _SCALE
    bw = jnp.maximum(x2 - x1, 1.0) / float(_PH)
    bh = jnp.maximum(y2 - y1, 1.0) / float(_PH)
    ya = _ybase(boxes_ref, k)

    wx = _interp_weights(x1, bw, _W, 0.0)
    wy = _interp_weights(y1, bh, _WIN, ya.astype(jnp.float32))

    f = feat_ref[b, :, pl.ds(ya, _WIN), :]
    mf = f * mwin_ref[k % _NBUF]

    a = jax.lax.dot_general(
        mf.reshape(_C * _WIN, _W), wx, (((1,), (0,)), ((), ())),
        preferred_element_type=jnp.float32).reshape(_C, _WIN, _SAMP)

    u = jax.lax.dot_general(
        a, wy, (((1,), (0,)), ((), ())),
        preferred_element_type=jnp.float32)

    s_ = jax.lax.broadcasted_iota(jnp.int32, (_SAMP * _SAMP, 56), 0)
    r_ = jax.lax.broadcasted_iota(jnp.int32, (_SAMP * _SAMP, 56), 1)
    bigpool = (((s_ % _SAMP) // 2 == r_ // _PH)
               & ((s_ // _SAMP) // 2 == r_ % _PH)).astype(jnp.float32) * 0.25

    w = jax.lax.dot_general(
        u.reshape(_C, _SAMP * _SAMP), bigpool, (((1,), (0,)), ((), ())),
        preferred_element_type=jnp.float32)
    out_ref[0] = w[:, :_PH * _PH]


@jax.jit
def kernel(features, boxes, masks):
    out = pl.pallas_call(
        _roi_kernel,
        grid=(_K,),
        in_specs=[
            pl.BlockSpec(memory_space=pltpu.SMEM),
            pl.BlockSpec((2, _C, _H, _W), lambda k: (0, 0, 0, 0)),
            pl.BlockSpec(memory_space=pltpu.MemorySpace.HBM),
        ],
        out_specs=pl.BlockSpec((1, _C, _PH * _PH), lambda k: (k, 0, 0)),
        out_shape=jax.ShapeDtypeStruct((_K, _C, _PH * _PH), jnp.float32),
        scratch_shapes=[
            pltpu.VMEM((_NBUF, _C, _WIN, _W), jnp.float32),
            pltpu.SemaphoreType.DMA((_NBUF,)),
        ],
        compiler_params=pltpu.CompilerParams(
            dimension_semantics=("arbitrary",),
        ),
    )(boxes, features, masks)
    return out.reshape(_K, _C, _PH, _PH)
